# baseline (device time: 13185 ns/iter reference)
import jax
import jax.numpy as jnp
from jax import lax
from jax.experimental import pallas as pl
from jax.experimental.pallas import tpu as pltpu

K = 16


def _topk_desc(work, k):
    neg = jnp.float32(-jnp.inf)
    vals = []
    for _ in range(k):
        m = jnp.max(work, axis=1, keepdims=True)
        vals.append(m)
        work = jnp.where(work == m, neg, work)
    return jnp.concatenate(vals, axis=1)


def _local_topk(x, k):
    rows, n = x.shape
    work = x.reshape(rows, n // 128, 128)
    neg = jnp.float32(-jnp.inf)
    summ = []
    for _ in range(3):
        m = jnp.max(work, axis=1)
        summ.append(m)
        work = jnp.where(work == m[:, None, :], neg, work)
    return _topk_desc(jnp.concatenate(summ, axis=1), k)


def _chunk_top3_tournament(w):
    hi1 = jnp.maximum(w[:, :16], w[:, 16:])
    lo1 = jnp.minimum(w[:, :16], w[:, 16:])
    hi2 = jnp.maximum(hi1[:, :8], hi1[:, 8:])
    lo2 = jnp.minimum(hi1[:, :8], hi1[:, 8:])
    hi3 = jnp.maximum(hi2[:, :4], hi2[:, 4:])
    lo3 = jnp.minimum(hi2[:, :4], hi2[:, 4:])
    hi4 = jnp.maximum(hi3[:, :2], hi3[:, 2:])
    lo4 = jnp.minimum(hi3[:, :2], hi3[:, 2:])
    r0 = jnp.maximum(hi4[:, 0], hi4[:, 1])
    r1 = jnp.minimum(hi4[:, 0], hi4[:, 1])
    r2 = jnp.maximum(lo4[:, 0], lo4[:, 1])
    r1, r2 = jnp.maximum(r1, r2), jnp.minimum(r1, r2)
    for lo in (lo3, lo2, lo1):
        m = jnp.max(lo, axis=1)
        r1n = jnp.maximum(r1, m)
        r2 = jnp.maximum(r2, jnp.minimum(r1, m))
        r1 = r1n
    return r0, r1, r2


def _local_topk_tournament(x, k):
    rows, n = x.shape
    r0, r1, r2 = _chunk_top3_tournament(x.reshape(rows, n // 128, 128))
    return _topk_desc(jnp.concatenate([r0, r1, r2], axis=1), k)


def _half(x):
    h = x.shape[1] // 2
    return x[:, :h], x[:, h:]


def _fold_max(x, width):
    while x.shape[1] > width:
        a, b = _half(x)
        x = jnp.maximum(a, b)
    return x


def _sorted3_fold(x):
    hi = x
    los = []
    while hi.shape[1] > 128:
        a, b = _half(hi)
        los.append(jnp.minimum(a, b))
        hi = jnp.maximum(a, b)
    r0 = hi
    r1 = los.pop()
    r2 = jnp.full_like(r1, -jnp.inf)
    for lo in reversed(los):
        m = _fold_max(lo, 128)
        r1n = jnp.maximum(r1, m)
        r2 = jnp.maximum(r2, jnp.minimum(r1, m))
        r1 = r1n
    return r0, r1, r2


def _merge_sorted3(a, b):
    m0, m3 = a[0], b[0]
    m1 = jnp.maximum(a[1], b[2])
    m2 = jnp.maximum(a[2], b[1])
    t0, t2 = jnp.maximum(m0, m2), jnp.minimum(m0, m2)
    t1, t3 = jnp.maximum(m1, m3), jnp.minimum(m1, m3)
    return (
        jnp.maximum(t0, t1),
        jnp.minimum(t0, t1),
        jnp.maximum(t2, t3),
    )


def _local_topk_tournament2d(x, k):
    r0, r1, r2 = _sorted3_fold(x)
    return _topk_desc(jnp.concatenate([r0, r1, r2], axis=1), k)


def _xor_shuffle_rows(x, d):
    k = x.shape[0]
    parts = []
    for s in range(0, k, 2 * d):
        parts.append(x[s + d : s + 2 * d, :])
        parts.append(x[s : s + d, :])
    return jnp.concatenate(parts, axis=0)


def _reverse_rows(x):
    k = x.shape[0]
    return jnp.concatenate([x[i : i + 1, :] for i in range(k - 1, -1, -1)], axis=0)


def _bitonic_merge_topk_t(at, bt):
    k = at.shape[0]
    m = jnp.maximum(at, _reverse_rows(bt))
    row = lax.broadcasted_iota(jnp.int32, m.shape, 0)
    d = k // 2
    while d >= 1:
        sw = _xor_shuffle_rows(m, d)
        hi = jnp.maximum(m, sw)
        lo = jnp.minimum(m, sw)
        m = jnp.where((row & d) == 0, hi, lo)
        d //= 2
    return m


N_BLK = 4


def kernel(x):
    m_rows, n_cols = x.shape
    nb = n_cols // N_BLK

    def body(x_ref, out_ref, acc_ref, cand_ref, recv_ref, send_sem, recv_sem):
        i = pl.program_id(0)
        my_x = lax.axis_index("x")
        my_y = lax.axis_index("y")
        peer = (my_x, 1 - my_y)

        @pl.when(i == 0)
        def _():
            barrier_sem = pltpu.get_barrier_semaphore()
            pl.semaphore_signal(
                barrier_sem, inc=1, device_id=peer,
                device_id_type=pl.DeviceIdType.MESH,
            )
            pl.semaphore_wait(barrier_sem, 1)

        s = _sorted3_fold(x_ref[:, :])

        @pl.when(i == 0)
        def _():
            for j in range(3):
                acc_ref[:, 128 * j : 128 * (j + 1)] = s[j]

        @pl.when(i != 0)
        def _():
            acc = tuple(acc_ref[:, 128 * j : 128 * (j + 1)] for j in range(3))
            merged = _merge_sorted3(acc, s)
            for j in range(3):
                acc_ref[:, 128 * j : 128 * (j + 1)] = merged[j]

        @pl.when(i == N_BLK - 1)
        def _():
            cand = _topk_desc(acc_ref[:, :], K)
            cand_t = cand.T
            cand_ref[:, :] = cand_t

            rdma = pltpu.make_async_remote_copy(
                src_ref=cand_ref,
                dst_ref=recv_ref,
                send_sem=send_sem,
                recv_sem=recv_sem,
                device_id=peer,
                device_id_type=pl.DeviceIdType.MESH,
            )
            rdma.start()
            rdma.wait_recv()

            out_ref[:, :] = _bitonic_merge_topk_t(cand_t, recv_ref[:, :]).T
            rdma.wait_send()

    return pl.pallas_call(
        body,
        grid=(N_BLK,),
        out_shape=jax.ShapeDtypeStruct((m_rows, K), jnp.float32),
        in_specs=[
            pl.BlockSpec((m_rows, nb), lambda i: (0, i), memory_space=pltpu.VMEM)
        ],
        out_specs=pl.BlockSpec(
            (m_rows, K), lambda i: (0, 0), memory_space=pltpu.VMEM
        ),
        scratch_shapes=[
            pltpu.VMEM((m_rows, 384), jnp.float32),
            pltpu.VMEM((K, m_rows), jnp.float32),
            pltpu.VMEM((K, m_rows), jnp.float32),
            pltpu.SemaphoreType.DMA,
            pltpu.SemaphoreType.DMA,
        ],
        compiler_params=pltpu.CompilerParams(collective_id=0),
    )(x)


# device time: 12969 ns/iter; 1.0167x vs baseline; 1.0167x over previous
import jax
import jax.numpy as jnp
from jax import lax
from jax.experimental import pallas as pl
from jax.experimental.pallas import tpu as pltpu

K = 16


def _topk_desc(work, k):
    neg = jnp.float32(-jnp.inf)
    vals = []
    for _ in range(k):
        m = jnp.max(work, axis=1, keepdims=True)
        vals.append(m)
        work = jnp.where(work == m, neg, work)
    return jnp.concatenate(vals, axis=1)


def _local_topk(x, k):
    rows, n = x.shape
    work = x.reshape(rows, n // 128, 128)
    neg = jnp.float32(-jnp.inf)
    summ = []
    for _ in range(3):
        m = jnp.max(work, axis=1)
        summ.append(m)
        work = jnp.where(work == m[:, None, :], neg, work)
    return _topk_desc(jnp.concatenate(summ, axis=1), k)


def _chunk_top3_tournament(w):
    hi1 = jnp.maximum(w[:, :16], w[:, 16:])
    lo1 = jnp.minimum(w[:, :16], w[:, 16:])
    hi2 = jnp.maximum(hi1[:, :8], hi1[:, 8:])
    lo2 = jnp.minimum(hi1[:, :8], hi1[:, 8:])
    hi3 = jnp.maximum(hi2[:, :4], hi2[:, 4:])
    lo3 = jnp.minimum(hi2[:, :4], hi2[:, 4:])
    hi4 = jnp.maximum(hi3[:, :2], hi3[:, 2:])
    lo4 = jnp.minimum(hi3[:, :2], hi3[:, 2:])
    r0 = jnp.maximum(hi4[:, 0], hi4[:, 1])
    r1 = jnp.minimum(hi4[:, 0], hi4[:, 1])
    r2 = jnp.maximum(lo4[:, 0], lo4[:, 1])
    r1, r2 = jnp.maximum(r1, r2), jnp.minimum(r1, r2)
    for lo in (lo3, lo2, lo1):
        m = jnp.max(lo, axis=1)
        r1n = jnp.maximum(r1, m)
        r2 = jnp.maximum(r2, jnp.minimum(r1, m))
        r1 = r1n
    return r0, r1, r2


def _local_topk_tournament(x, k):
    rows, n = x.shape
    r0, r1, r2 = _chunk_top3_tournament(x.reshape(rows, n // 128, 128))
    return _topk_desc(jnp.concatenate([r0, r1, r2], axis=1), k)


def _half(x):
    h = x.shape[1] // 2
    return x[:, :h], x[:, h:]


def _fold_max(x, width):
    while x.shape[1] > width:
        a, b = _half(x)
        x = jnp.maximum(a, b)
    return x


def _sorted3_fold(x):
    hi = x
    los = []
    while hi.shape[1] > 128:
        a, b = _half(hi)
        los.append(jnp.minimum(a, b))
        hi = jnp.maximum(a, b)
    r0 = hi
    r1 = los.pop()
    r2 = jnp.full_like(r1, -jnp.inf)
    for lo in reversed(los):
        m = _fold_max(lo, 128)
        r1n = jnp.maximum(r1, m)
        r2 = jnp.maximum(r2, jnp.minimum(r1, m))
        r1 = r1n
    return r0, r1, r2


def _merge_sorted3(a, b):
    m0, m3 = a[0], b[0]
    m1 = jnp.maximum(a[1], b[2])
    m2 = jnp.maximum(a[2], b[1])
    t0, t2 = jnp.maximum(m0, m2), jnp.minimum(m0, m2)
    t1, t3 = jnp.maximum(m1, m3), jnp.minimum(m1, m3)
    return (
        jnp.maximum(t0, t1),
        jnp.minimum(t0, t1),
        jnp.maximum(t2, t3),
    )


def _local_topk_tournament2d(x, k):
    r0, r1, r2 = _sorted3_fold(x)
    return _topk_desc(jnp.concatenate([r0, r1, r2], axis=1), k)


def _xor_shuffle_rows(x, d):
    k = x.shape[0]
    parts = []
    for s in range(0, k, 2 * d):
        parts.append(x[s + d : s + 2 * d, :])
        parts.append(x[s : s + d, :])
    return jnp.concatenate(parts, axis=0)


def _reverse_rows(x):
    k = x.shape[0]
    return jnp.concatenate([x[i : i + 1, :] for i in range(k - 1, -1, -1)], axis=0)


def _bitonic_merge_topk_t(at, bt):
    k = at.shape[0]
    m = jnp.maximum(at, _reverse_rows(bt))
    row = lax.broadcasted_iota(jnp.int32, m.shape, 0)
    d = k // 2
    while d >= 1:
        sw = _xor_shuffle_rows(m, d)
        hi = jnp.maximum(m, sw)
        lo = jnp.minimum(m, sw)
        m = jnp.where((row & d) == 0, hi, lo)
        d //= 2
    return m


def kernel(x):
    m_rows, n_cols = x.shape
    half = m_rows // 2

    def body(
        x_ref, out_ref, xv_ref, yfull_ref, yrecv_ref,
        copy_sem, x_send_sem, x_recv_sem, y_send_sem, y_recv_sem,
    ):
        my_x = lax.axis_index("x")
        my_y = lax.axis_index("y")
        x_peer = (1 - my_x, my_y)
        y_peer = (my_x, 1 - my_y)

        barrier_sem = pltpu.get_barrier_semaphore()
        for nbr in (x_peer, y_peer):
            pl.semaphore_signal(
                barrier_sem, inc=1, device_id=nbr,
                device_id_type=pl.DeviceIdType.MESH,
            )
        pl.semaphore_wait(barrier_sem, 2)

        copy = pltpu.make_async_copy(
            x_ref.at[pl.ds(my_x * half, half), :], xv_ref, copy_sem
        )
        copy.start()
        copy.wait()

        cand = _local_topk_tournament2d(xv_ref[:, :], K)
        yfull_ref[:, pl.ds(my_x * half, half)] = cand.T

        x_rdma = pltpu.make_async_remote_copy(
            src_ref=yfull_ref.at[:, pl.ds(my_x * half, half)],
            dst_ref=yfull_ref.at[:, pl.ds(my_x * half, half)],
            send_sem=x_send_sem,
            recv_sem=x_recv_sem,
            device_id=x_peer,
            device_id_type=pl.DeviceIdType.MESH,
        )
        x_rdma.start()
        x_rdma.wait_recv()

        y_rdma = pltpu.make_async_remote_copy(
            src_ref=yfull_ref,
            dst_ref=yrecv_ref,
            send_sem=y_send_sem,
            recv_sem=y_recv_sem,
            device_id=y_peer,
            device_id_type=pl.DeviceIdType.MESH,
        )
        y_rdma.start()
        y_rdma.wait_recv()

        out_ref[:, :] = _bitonic_merge_topk_t(
            yfull_ref[:, :], yrecv_ref[:, :]
        ).T
        x_rdma.wait_send()
        y_rdma.wait_send()

    return pl.pallas_call(
        body,
        out_shape=jax.ShapeDtypeStruct((m_rows, K), jnp.float32),
        in_specs=[pl.BlockSpec(memory_space=pl.ANY)],
        out_specs=pl.BlockSpec(memory_space=pltpu.VMEM),
        scratch_shapes=[
            pltpu.VMEM((half, n_cols), jnp.float32),
            pltpu.VMEM((K, m_rows), jnp.float32),
            pltpu.VMEM((K, m_rows), jnp.float32),
            pltpu.SemaphoreType.DMA,
            pltpu.SemaphoreType.DMA,
            pltpu.SemaphoreType.DMA,
            pltpu.SemaphoreType.DMA,
            pltpu.SemaphoreType.DMA,
        ],
        compiler_params=pltpu.CompilerParams(collective_id=0),
    )(x)


# device time: 10136 ns/iter; 1.3008x vs baseline; 1.2795x over previous
import jax
import jax.numpy as jnp
from jax import lax
from jax.experimental import pallas as pl
from jax.experimental.pallas import tpu as pltpu

K = 16


def _topk_desc(work, k):
    neg = jnp.float32(-jnp.inf)
    vals = []
    for _ in range(k):
        m = jnp.max(work, axis=1, keepdims=True)
        vals.append(m)
        work = jnp.where(work == m, neg, work)
    return jnp.concatenate(vals, axis=1)


def _local_topk(x, k):
    rows, n = x.shape
    work = x.reshape(rows, n // 128, 128)
    neg = jnp.float32(-jnp.inf)
    summ = []
    for _ in range(3):
        m = jnp.max(work, axis=1)
        summ.append(m)
        work = jnp.where(work == m[:, None, :], neg, work)
    return _topk_desc(jnp.concatenate(summ, axis=1), k)


def _chunk_top3_tournament(w):
    hi1 = jnp.maximum(w[:, :16], w[:, 16:])
    lo1 = jnp.minimum(w[:, :16], w[:, 16:])
    hi2 = jnp.maximum(hi1[:, :8], hi1[:, 8:])
    lo2 = jnp.minimum(hi1[:, :8], hi1[:, 8:])
    hi3 = jnp.maximum(hi2[:, :4], hi2[:, 4:])
    lo3 = jnp.minimum(hi2[:, :4], hi2[:, 4:])
    hi4 = jnp.maximum(hi3[:, :2], hi3[:, 2:])
    lo4 = jnp.minimum(hi3[:, :2], hi3[:, 2:])
    r0 = jnp.maximum(hi4[:, 0], hi4[:, 1])
    r1 = jnp.minimum(hi4[:, 0], hi4[:, 1])
    r2 = jnp.maximum(lo4[:, 0], lo4[:, 1])
    r1, r2 = jnp.maximum(r1, r2), jnp.minimum(r1, r2)
    for lo in (lo3, lo2, lo1):
        m = jnp.max(lo, axis=1)
        r1n = jnp.maximum(r1, m)
        r2 = jnp.maximum(r2, jnp.minimum(r1, m))
        r1 = r1n
    return r0, r1, r2


def _local_topk_tournament(x, k):
    rows, n = x.shape
    r0, r1, r2 = _chunk_top3_tournament(x.reshape(rows, n // 128, 128))
    return _topk_desc(jnp.concatenate([r0, r1, r2], axis=1), k)


def _half(x):
    h = x.shape[1] // 2
    return x[:, :h], x[:, h:]


def _fold_max(x, width):
    while x.shape[1] > width:
        a, b = _half(x)
        x = jnp.maximum(a, b)
    return x


def _sorted3_fold(x):
    hi = x
    los = []
    while hi.shape[1] > 128:
        a, b = _half(hi)
        los.append(jnp.minimum(a, b))
        hi = jnp.maximum(a, b)
    r0 = hi
    r1 = los.pop()
    r2 = jnp.full_like(r1, -jnp.inf)
    for lo in reversed(los):
        m = _fold_max(lo, 128)
        r1n = jnp.maximum(r1, m)
        r2 = jnp.maximum(r2, jnp.minimum(r1, m))
        r1 = r1n
    return r0, r1, r2


def _merge_sorted3(a, b):
    m0, m3 = a[0], b[0]
    m1 = jnp.maximum(a[1], b[2])
    m2 = jnp.maximum(a[2], b[1])
    t0, t2 = jnp.maximum(m0, m2), jnp.minimum(m0, m2)
    t1, t3 = jnp.maximum(m1, m3), jnp.minimum(m1, m3)
    return (
        jnp.maximum(t0, t1),
        jnp.minimum(t0, t1),
        jnp.maximum(t2, t3),
    )


def _local_topk_tournament2d(x, k):
    r0, r1, r2 = _sorted3_fold(x)
    return _topk_desc(jnp.concatenate([r0, r1, r2], axis=1), k)


def _bitonic_sort_stack(s):
    L = len(s)
    d = L // 2
    while d >= 1:
        ns = list(s)
        for i in range(L):
            if i & d == 0:
                ns[i] = jnp.maximum(s[i], s[i ^ d])
                ns[i ^ d] = jnp.minimum(s[i], s[i ^ d])
        s = ns
        d //= 2
    return s


def _topk16_tournament_t(r0, r1, r2):
    pad = jnp.full_like(r0, -jnp.inf)
    stack = [r0.T, r1.T, r2.T, pad.T]
    w = 128
    while w > 1:
        h = w // 2
        a = [s[:h] for s in stack]
        b = [s[h:] for s in stack]
        L = len(stack)
        if 2 * L <= 16:
            stack = _bitonic_sort_stack(a + list(reversed(b)))
        else:
            p = [jnp.full_like(a[0], -jnp.inf)] * (16 - L)
            a = a + p
            b = b + p
            m = [jnp.maximum(a[i], b[15 - i]) for i in range(16)]
            stack = _bitonic_sort_stack(m)
        w = h
    return jnp.concatenate(stack, axis=0)


def _xor_shuffle_rows(x, d):
    k = x.shape[0]
    parts = []
    for s in range(0, k, 2 * d):
        parts.append(x[s + d : s + 2 * d, :])
        parts.append(x[s : s + d, :])
    return jnp.concatenate(parts, axis=0)


def _reverse_rows(x):
    k = x.shape[0]
    return jnp.concatenate([x[i : i + 1, :] for i in range(k - 1, -1, -1)], axis=0)


def _bitonic_merge_topk_t(at, bt):
    k = at.shape[0]
    m = jnp.maximum(at, _reverse_rows(bt))
    row = lax.broadcasted_iota(jnp.int32, m.shape, 0)
    d = k // 2
    while d >= 1:
        sw = _xor_shuffle_rows(m, d)
        hi = jnp.maximum(m, sw)
        lo = jnp.minimum(m, sw)
        m = jnp.where((row & d) == 0, hi, lo)
        d //= 2
    return m


def kernel(x):
    m_rows, n_cols = x.shape

    def body(x_ref, out_ref, cand_ref, recv_ref, send_sem, recv_sem):
        my_x = lax.axis_index("x")
        my_y = lax.axis_index("y")
        peer = (my_x, 1 - my_y)

        barrier_sem = pltpu.get_barrier_semaphore()
        pl.semaphore_signal(
            barrier_sem, inc=1, device_id=peer,
            device_id_type=pl.DeviceIdType.MESH,
        )
        pl.semaphore_wait(barrier_sem, 1)

        r0, r1, r2 = _sorted3_fold(x_ref[:, :])
        cand_t = _topk16_tournament_t(r0, r1, r2)
        cand_ref[:, :] = cand_t

        rdma = pltpu.make_async_remote_copy(
            src_ref=cand_ref,
            dst_ref=recv_ref,
            send_sem=send_sem,
            recv_sem=recv_sem,
            device_id=peer,
            device_id_type=pl.DeviceIdType.MESH,
        )
        rdma.start()
        rdma.wait_recv()

        out_ref[:, :] = _bitonic_merge_topk_t(cand_t, recv_ref[:, :]).T
        rdma.wait_send()

    return pl.pallas_call(
        body,
        out_shape=jax.ShapeDtypeStruct((m_rows, K), jnp.float32),
        in_specs=[pl.BlockSpec(memory_space=pltpu.VMEM)],
        out_specs=pl.BlockSpec(memory_space=pltpu.VMEM),
        scratch_shapes=[
            pltpu.VMEM((K, m_rows), jnp.float32),
            pltpu.VMEM((K, m_rows), jnp.float32),
            pltpu.SemaphoreType.DMA,
            pltpu.SemaphoreType.DMA,
        ],
        compiler_params=pltpu.CompilerParams(collective_id=0),
    )(x)
